# Initial kernel scaffold; baseline (speedup 1.0000x reference)
import functools

import jax
import jax.numpy as jnp
from jax import lax
from jax.experimental import pallas as pl
from jax.experimental.pallas import tpu as pltpu
from jax.experimental.pallas import tpu_sc as plsc

N = 10000
E = 160000
D = 256
H = 128
NS = 16
EPT = E // (NS)
K = 80
NCHUNK = EPT // K
RPT = N // NS
ZR = 25


def _sc_aggregate(x2, src, dst):
    mesh = plsc.VectorSubcoreMesh(core_axis_name="c", subcore_axis_name="s")

    @functools.partial(
        pl.kernel,
        mesh=mesh,
        out_type=jax.ShapeDtypeStruct((2, N, H), jnp.float32),
        scratch_types=[
            pltpu.VMEM((K,), jnp.int32),
            pltpu.VMEM((K,), jnp.int32),
            pltpu.VMEM((K,), jnp.int32),
            pltpu.VMEM((K, H), jnp.float32),
            pltpu.VMEM((ZR, H), jnp.float32),
            pltpu.VMEM_SHARED((N, H), jnp.float32),
            pltpu.SemaphoreType.DMA,
        ],
    )
    def agg(x2_hbm, src_hbm, dst_hbm, out_hbm, src_v, idx_v, dst_v, rows_v,
            zero_v, accum_sh, sem):
        c = lax.axis_index("c")
        s = lax.axis_index("s")

        zvec = jnp.zeros((16,), jnp.float32)
        for i in range(ZR):
            for j in range(H // 16):
                zero_v[i, pl.ds(j * 16, 16)] = zvec

        def zero_body(t, carry):
            pltpu.sync_copy(zero_v, accum_sh.at[pl.ds(s * RPT + t * ZR, ZR)])
            return carry

        lax.fori_loop(0, RPT // ZR, zero_body, 0)
        plsc.subcore_barrier()

        def edge_body(t, carry):
            base = s * EPT + t * K
            pltpu.sync_copy(src_hbm.at[pl.ds(base, K)], src_v)
            pltpu.sync_copy(dst_hbm.at[pl.ds(base, K)], dst_v)
            for j in range(K // 16):
                sl = pl.ds(j * 16, 16)
                idx_v[sl] = src_v[sl] * 2 + c
            pltpu.async_copy(x2_hbm.at[idx_v], rows_v, sem).wait()
            pltpu.sync_copy(rows_v, accum_sh.at[dst_v], add=True)
            return carry

        lax.fori_loop(0, NCHUNK, edge_body, 0)
        plsc.subcore_barrier()

        pltpu.sync_copy(accum_sh.at[pl.ds(s * RPT, RPT)],
                        out_hbm.at[c, pl.ds(s * RPT, RPT)])

    return agg(x2, src, dst)


def _mlp_body(x_ref, a0_ref, a1_ref, W1_ref, b1_ref, g1_ref, be1_ref,
              W2_ref, b2_ref, g2_ref, be2_ref, o_ref):
    h = x_ref[...] + jnp.concatenate([a0_ref[0], a1_ref[0]], axis=1)

    h = jnp.dot(h, W1_ref[...], preferred_element_type=jnp.float32) + b1_ref[...]
    mu = jnp.mean(h, axis=-1, keepdims=True)
    var = jnp.mean((h - mu) ** 2, axis=-1, keepdims=True)
    h = (h - mu) / jnp.sqrt(var + 1e-5) * g1_ref[...] + be1_ref[...]
    h = 0.5 * h * (1.0 + lax.erf(h / jnp.sqrt(2.0).astype(jnp.float32)))

    h = jnp.dot(h, W2_ref[...], preferred_element_type=jnp.float32) + b2_ref[...]
    mu = jnp.mean(h, axis=-1, keepdims=True)
    var = jnp.mean((h - mu) ** 2, axis=-1, keepdims=True)
    h = (h - mu) / jnp.sqrt(var + 1e-5) * g2_ref[...] + be2_ref[...]
    h = 0.5 * h * (1.0 + lax.erf(h / jnp.sqrt(2.0).astype(jnp.float32)))

    o_ref[...] = h


def _tc_mlp(x, aggr2, W1, b1, g1, be1, W2, b2, g2, be2):
    BN = 400
    grid = (N // BN,)
    vec = lambda: pl.BlockSpec((1, D), lambda i: (0, 0))
    mat = lambda: pl.BlockSpec((D, D), lambda i: (0, 0))
    return pl.pallas_call(
        _mlp_body,
        grid=grid,
        in_specs=[
            pl.BlockSpec((BN, D), lambda i: (i, 0)),
            pl.BlockSpec((1, BN, H), lambda i: (0, i, 0)),
            pl.BlockSpec((1, BN, H), lambda i: (1, i, 0)),
            mat(), vec(), vec(), vec(),
            mat(), vec(), vec(), vec(),
        ],
        out_specs=pl.BlockSpec((BN, D), lambda i: (i, 0)),
        out_shape=jax.ShapeDtypeStruct((N, D), jnp.float32),
    )(x, aggr2, aggr2, W1, b1.reshape(1, D), g1.reshape(1, D),
      be1.reshape(1, D), W2, b2.reshape(1, D), g2.reshape(1, D),
      be2.reshape(1, D))


def kernel(x, edge_index, W1, b1, g1, be1, W2, b2, g2, be2):
    x2 = x.reshape(2 * N, H)
    src = edge_index[0]
    dst = edge_index[1]
    aggr2 = _sc_aggregate(x2, src, dst)
    return _tc_mlp(x, aggr2, W1, b1, g1, be1, W2, b2, g2, be2)


# speedup vs baseline: 3.7403x; 3.7403x over previous
import functools

import jax
import jax.numpy as jnp
from jax import lax
from jax.experimental import pallas as pl
from jax.experimental.pallas import tpu as pltpu
from jax.experimental.pallas import tpu_sc as plsc

N = 10000
E = 160000
D = 256
H = 128
NS = 16
EPT = E // (NS)
K = 80
NCHUNK = EPT // K
NP = 10240
RPT = NP // NS
ZR = 32


def _sc_aggregate(x2, src, dst):
    mesh = plsc.VectorSubcoreMesh(core_axis_name="c", subcore_axis_name="s")

    @functools.partial(
        pl.kernel,
        mesh=mesh,
        out_type=jax.ShapeDtypeStruct((2, NP, H), jnp.float32),
        scratch_types=[
            pltpu.VMEM((K,), jnp.int32),
            pltpu.VMEM((K,), jnp.int32),
            pltpu.VMEM((K,), jnp.int32),
            pltpu.VMEM((K, H), jnp.float32),
            pltpu.VMEM((ZR, H), jnp.float32),
            pltpu.VMEM_SHARED((NP, H), jnp.float32),
            pltpu.SemaphoreType.DMA,
        ],
    )
    def agg(x2_hbm, src_hbm, dst_hbm, out_hbm, src_v, idx_v, dst_v, rows_v,
            zero_v, accum_sh, sem):
        c = lax.axis_index("c")
        s = lax.axis_index("s")

        zvec = jnp.zeros((16,), jnp.float32)
        for i in range(ZR):
            for j in range(H // 16):
                zero_v[i, pl.ds(j * 16, 16)] = zvec

        def zero_body(t, carry):
            pltpu.sync_copy(zero_v, accum_sh.at[pl.ds(s * RPT + t * ZR, ZR)])
            return carry

        lax.fori_loop(0, RPT // ZR, zero_body, 0)
        plsc.subcore_barrier()

        def edge_body(t, carry):
            base = s * EPT + t * K
            pltpu.sync_copy(src_hbm.at[pl.ds(base, K)], src_v)
            pltpu.sync_copy(dst_hbm.at[pl.ds(base, K)], dst_v)
            for j in range(K // 16):
                sl = pl.ds(j * 16, 16)
                idx_v[sl] = src_v[sl] * 2 + c
            pltpu.async_copy(x2_hbm.at[idx_v], rows_v, sem).wait()
            pltpu.sync_copy(rows_v, accum_sh.at[dst_v], add=True)
            return carry

        lax.fori_loop(0, NCHUNK, edge_body, 0)
        plsc.subcore_barrier()

        pltpu.sync_copy(accum_sh.at[pl.ds(s * RPT, RPT)],
                        out_hbm.at[c, pl.ds(s * RPT, RPT)])

    return agg(x2, src, dst)


def _mlp_body(x_ref, a0_ref, a1_ref, W1_ref, b1_ref, g1_ref, be1_ref,
              W2_ref, b2_ref, g2_ref, be2_ref, o_ref):
    h = x_ref[...] + jnp.concatenate([a0_ref[0], a1_ref[0]], axis=1)

    h = jnp.dot(h, W1_ref[...], preferred_element_type=jnp.float32) + b1_ref[...]
    mu = jnp.mean(h, axis=-1, keepdims=True)
    var = jnp.mean((h - mu) ** 2, axis=-1, keepdims=True)
    h = (h - mu) / jnp.sqrt(var + 1e-5) * g1_ref[...] + be1_ref[...]
    h = 0.5 * h * (1.0 + lax.erf(h / jnp.sqrt(2.0).astype(jnp.float32)))

    h = jnp.dot(h, W2_ref[...], preferred_element_type=jnp.float32) + b2_ref[...]
    mu = jnp.mean(h, axis=-1, keepdims=True)
    var = jnp.mean((h - mu) ** 2, axis=-1, keepdims=True)
    h = (h - mu) / jnp.sqrt(var + 1e-5) * g2_ref[...] + be2_ref[...]
    h = 0.5 * h * (1.0 + lax.erf(h / jnp.sqrt(2.0).astype(jnp.float32)))

    o_ref[...] = h


def _tc_mlp(x, aggr2, W1, b1, g1, be1, W2, b2, g2, be2):
    BN = 400
    grid = (N // BN,)
    vec = lambda: pl.BlockSpec((1, D), lambda i: (0, 0))
    mat = lambda: pl.BlockSpec((D, D), lambda i: (0, 0))
    return pl.pallas_call(
        _mlp_body,
        grid=grid,
        in_specs=[
            pl.BlockSpec((BN, D), lambda i: (i, 0)),
            pl.BlockSpec((1, BN, H), lambda i: (0, i, 0)),
            pl.BlockSpec((1, BN, H), lambda i: (1, i, 0)),
            mat(), vec(), vec(), vec(),
            mat(), vec(), vec(), vec(),
        ],
        out_specs=pl.BlockSpec((BN, D), lambda i: (i, 0)),
        out_shape=jax.ShapeDtypeStruct((N, D), jnp.float32),
    )(x, aggr2, aggr2, W1, b1.reshape(1, D), g1.reshape(1, D),
      be1.reshape(1, D), W2, b2.reshape(1, D), g2.reshape(1, D),
      be2.reshape(1, D))


def kernel(x, edge_index, W1, b1, g1, be1, W2, b2, g2, be2):
    x2 = x.reshape(2 * N, H)
    src = edge_index[0]
    dst = edge_index[1]
    aggr2 = _sc_aggregate(x2, src, dst)
    return _tc_mlp(x, aggr2, W1, b1, g1, be1, W2, b2, g2, be2)


# speedup vs baseline: 5.6311x; 1.5055x over previous
import functools

import jax
import jax.numpy as jnp
from jax import lax
from jax.experimental import pallas as pl
from jax.experimental.pallas import tpu as pltpu
from jax.experimental.pallas import tpu_sc as plsc

N = 10000
E = 160000
D = 256
H = 128
NS = 16
EPT = E // (NS)
K = 80
NCHUNK = EPT // K
NB = 4
NGRP = NCHUNK // NB
NTAIL = NCHUNK - NGRP * NB
NP = 10240
RPT = NP // NS


def _sc_aggregate(x2, src, dst):
    mesh = plsc.VectorSubcoreMesh(core_axis_name="c", subcore_axis_name="s")

    @functools.partial(
        pl.kernel,
        mesh=mesh,
        out_type=jax.ShapeDtypeStruct((2, NP, H), jnp.float32),
        scratch_types=(
            [pltpu.VMEM((K,), jnp.int32)] * NB
            + [pltpu.VMEM((K,), jnp.int32)] * NB
            + [pltpu.VMEM((K,), jnp.int32)] * NB
            + [pltpu.VMEM((K, H), jnp.float32)] * NB
            + [pltpu.VMEM_SHARED((NP, H), jnp.float32)]
            + [pltpu.SemaphoreType.DMA] * NB
        ),
    )
    def agg(x2_hbm, src_hbm, dst_hbm, out_hbm, *scratch):
        src_v = scratch[0:NB]
        idx_v = scratch[NB:2 * NB]
        dst_v = scratch[2 * NB:3 * NB]
        rows_v = scratch[3 * NB:4 * NB]
        accum_sh = scratch[4 * NB]
        semg = scratch[4 * NB + 1:4 * NB + 1 + NB]
        c = lax.axis_index("c")
        s = lax.axis_index("s")

        zvec = jnp.zeros((16,), jnp.float32)
        for i in range(K):
            for j in range(H // 16):
                rows_v[0][i, pl.ds(j * 16, 16)] = zvec

        def zero_body(t, carry):
            pltpu.sync_copy(rows_v[0], accum_sh.at[pl.ds(s * RPT + t * K, K)])
            return carry

        lax.fori_loop(0, RPT // K, zero_body, 0)
        plsc.subcore_barrier()

        def grp_body(g, carry):
            hg = []
            for b in range(NB):
                base = s * EPT + (g * NB + b) * K
                pltpu.sync_copy(src_hbm.at[pl.ds(base, K)], src_v[b])
                pltpu.sync_copy(dst_hbm.at[pl.ds(base, K)], dst_v[b])
                for j in range(K // 16):
                    sl = pl.ds(j * 16, 16)
                    idx_v[b][sl] = src_v[b][sl] * 2 + c
                hg.append(pltpu.async_copy(x2_hbm.at[idx_v[b]],
                                           rows_v[b], semg[b]))
            for b in range(NB):
                hg[b].wait()
                pltpu.sync_copy(rows_v[b], accum_sh.at[dst_v[b]],
                                add=True)
            return carry

        lax.fori_loop(0, NGRP, grp_body, 0)

        for t in range(NGRP * NB, NCHUNK):
            base = s * EPT + t * K
            pltpu.sync_copy(src_hbm.at[pl.ds(base, K)], src_v[0])
            pltpu.sync_copy(dst_hbm.at[pl.ds(base, K)], dst_v[0])
            for j in range(K // 16):
                sl = pl.ds(j * 16, 16)
                idx_v[0][sl] = src_v[0][sl] * 2 + c
            pltpu.async_copy(x2_hbm.at[idx_v[0]], rows_v[0], semg[0]).wait()
            pltpu.sync_copy(rows_v[0], accum_sh.at[dst_v[0]], add=True)
        plsc.subcore_barrier()

        pltpu.sync_copy(accum_sh.at[pl.ds(s * RPT, RPT)],
                        out_hbm.at[c, pl.ds(s * RPT, RPT)])

    return agg(x2, src, dst)


def _mlp_body(x_ref, a0_ref, a1_ref, W1_ref, b1_ref, g1_ref, be1_ref,
              W2_ref, b2_ref, g2_ref, be2_ref, o_ref):
    h = x_ref[...] + jnp.concatenate([a0_ref[0], a1_ref[0]], axis=1)

    h = jnp.dot(h, W1_ref[...], preferred_element_type=jnp.float32) + b1_ref[...]
    mu = jnp.mean(h, axis=-1, keepdims=True)
    var = jnp.mean((h - mu) ** 2, axis=-1, keepdims=True)
    h = (h - mu) / jnp.sqrt(var + 1e-5) * g1_ref[...] + be1_ref[...]
    h = 0.5 * h * (1.0 + lax.erf(h / jnp.sqrt(2.0).astype(jnp.float32)))

    h = jnp.dot(h, W2_ref[...], preferred_element_type=jnp.float32) + b2_ref[...]
    mu = jnp.mean(h, axis=-1, keepdims=True)
    var = jnp.mean((h - mu) ** 2, axis=-1, keepdims=True)
    h = (h - mu) / jnp.sqrt(var + 1e-5) * g2_ref[...] + be2_ref[...]
    h = 0.5 * h * (1.0 + lax.erf(h / jnp.sqrt(2.0).astype(jnp.float32)))

    o_ref[...] = h


def _tc_mlp(x, aggr2, W1, b1, g1, be1, W2, b2, g2, be2):
    BN = 400
    grid = (N // BN,)
    vec = lambda: pl.BlockSpec((1, D), lambda i: (0, 0))
    mat = lambda: pl.BlockSpec((D, D), lambda i: (0, 0))
    return pl.pallas_call(
        _mlp_body,
        grid=grid,
        in_specs=[
            pl.BlockSpec((BN, D), lambda i: (i, 0)),
            pl.BlockSpec((1, BN, H), lambda i: (0, i, 0)),
            pl.BlockSpec((1, BN, H), lambda i: (1, i, 0)),
            mat(), vec(), vec(), vec(),
            mat(), vec(), vec(), vec(),
        ],
        out_specs=pl.BlockSpec((BN, D), lambda i: (i, 0)),
        out_shape=jax.ShapeDtypeStruct((N, D), jnp.float32),
    )(x, aggr2, aggr2, W1, b1.reshape(1, D), g1.reshape(1, D),
      be1.reshape(1, D), W2, b2.reshape(1, D), g2.reshape(1, D),
      be2.reshape(1, D))


def kernel(x, edge_index, W1, b1, g1, be1, W2, b2, g2, be2):
    x2 = x.reshape(2 * N, H)
    src = edge_index[0]
    dst = edge_index[1]
    aggr2 = _sc_aggregate(x2, src, dst)
    return _tc_mlp(x, aggr2, W1, b1, g1, be1, W2, b2, g2, be2)


# speedup vs baseline: 5.7403x; 1.0194x over previous
import functools

import jax
import jax.numpy as jnp
from jax import lax
from jax.experimental import pallas as pl
from jax.experimental.pallas import tpu as pltpu
from jax.experimental.pallas import tpu_sc as plsc

N = 10000
E = 160000
D = 256
H = 128
NS = 16
EPT = E // (NS)
K = 80
NCHUNK = EPT // K
NB = 4
NGRP = NCHUNK // NB
NTAIL = NCHUNK - NGRP * NB
NP = 10240
RPT = NP // NS


def _sc_aggregate(x2, src, dst):
    mesh = plsc.VectorSubcoreMesh(core_axis_name="c", subcore_axis_name="s")

    @functools.partial(
        pl.kernel,
        mesh=mesh,
        out_type=jax.ShapeDtypeStruct((2, NP, H), jnp.float32),
        scratch_types=(
            [pltpu.VMEM((K,), jnp.int32)] * NB
            + [pltpu.VMEM((K,), jnp.int32)] * NB
            + [pltpu.VMEM((K,), jnp.int32)] * NB
            + [pltpu.VMEM((K, H), jnp.float32)] * NB
            + [pltpu.VMEM_SHARED((NP, H), jnp.float32)]
            + [pltpu.SemaphoreType.DMA] * (2 * NB)
        ),
    )
    def agg(x2_hbm, src_hbm, dst_hbm, out_hbm, *scratch):
        src_v = scratch[0:NB]
        idx_v = scratch[NB:2 * NB]
        dst_v = scratch[2 * NB:3 * NB]
        rows_v = scratch[3 * NB:4 * NB]
        accum_sh = scratch[4 * NB]
        semg = scratch[4 * NB + 1:4 * NB + 1 + NB]
        semsc = scratch[4 * NB + 1 + NB:4 * NB + 1 + 2 * NB]
        c = lax.axis_index("c")
        s = lax.axis_index("s")

        zvec = jnp.zeros((16,), jnp.float32)
        for i in range(K):
            for j in range(H // 16):
                rows_v[0][i, pl.ds(j * 16, 16)] = zvec

        def zero_body(t, carry):
            pltpu.sync_copy(rows_v[0], accum_sh.at[pl.ds(s * RPT + t * K, K)])
            return carry

        lax.fori_loop(0, RPT // K, zero_body, 0)
        plsc.subcore_barrier()

        def grp_body(g, carry):
            hg = []
            for b in range(NB):
                base = s * EPT + (g * NB + b) * K
                pltpu.sync_copy(src_hbm.at[pl.ds(base, K)], src_v[b])
                pltpu.sync_copy(dst_hbm.at[pl.ds(base, K)], dst_v[b])
                for j in range(K // 16):
                    sl = pl.ds(j * 16, 16)
                    idx_v[b][sl] = src_v[b][sl] * 2 + c
                hg.append(pltpu.async_copy(x2_hbm.at[idx_v[b]],
                                           rows_v[b], semg[b]))
            hs = []
            for b in range(NB):
                hg[b].wait()
                hs.append(pltpu.async_copy(rows_v[b], accum_sh.at[dst_v[b]],
                                           semsc[b], add=True))
            for b in range(NB):
                hs[b].wait()
            return carry

        lax.fori_loop(0, NGRP, grp_body, 0)

        for t in range(NGRP * NB, NCHUNK):
            base = s * EPT + t * K
            pltpu.sync_copy(src_hbm.at[pl.ds(base, K)], src_v[0])
            pltpu.sync_copy(dst_hbm.at[pl.ds(base, K)], dst_v[0])
            for j in range(K // 16):
                sl = pl.ds(j * 16, 16)
                idx_v[0][sl] = src_v[0][sl] * 2 + c
            pltpu.async_copy(x2_hbm.at[idx_v[0]], rows_v[0], semg[0]).wait()
            pltpu.sync_copy(rows_v[0], accum_sh.at[dst_v[0]], add=True)
        plsc.subcore_barrier()

        pltpu.sync_copy(accum_sh.at[pl.ds(s * RPT, RPT)],
                        out_hbm.at[c, pl.ds(s * RPT, RPT)])

    return agg(x2, src, dst)


def _mlp_body(x_ref, a0_ref, a1_ref, W1_ref, b1_ref, g1_ref, be1_ref,
              W2_ref, b2_ref, g2_ref, be2_ref, o_ref):
    h = x_ref[...] + jnp.concatenate([a0_ref[0], a1_ref[0]], axis=1)

    h = jnp.dot(h, W1_ref[...], preferred_element_type=jnp.float32) + b1_ref[...]
    mu = jnp.mean(h, axis=-1, keepdims=True)
    var = jnp.mean((h - mu) ** 2, axis=-1, keepdims=True)
    h = (h - mu) / jnp.sqrt(var + 1e-5) * g1_ref[...] + be1_ref[...]
    h = 0.5 * h * (1.0 + lax.erf(h / jnp.sqrt(2.0).astype(jnp.float32)))

    h = jnp.dot(h, W2_ref[...], preferred_element_type=jnp.float32) + b2_ref[...]
    mu = jnp.mean(h, axis=-1, keepdims=True)
    var = jnp.mean((h - mu) ** 2, axis=-1, keepdims=True)
    h = (h - mu) / jnp.sqrt(var + 1e-5) * g2_ref[...] + be2_ref[...]
    h = 0.5 * h * (1.0 + lax.erf(h / jnp.sqrt(2.0).astype(jnp.float32)))

    o_ref[...] = h


def _tc_mlp(x, aggr2, W1, b1, g1, be1, W2, b2, g2, be2):
    BN = 400
    grid = (N // BN,)
    vec = lambda: pl.BlockSpec((1, D), lambda i: (0, 0))
    mat = lambda: pl.BlockSpec((D, D), lambda i: (0, 0))
    return pl.pallas_call(
        _mlp_body,
        grid=grid,
        in_specs=[
            pl.BlockSpec((BN, D), lambda i: (i, 0)),
            pl.BlockSpec((1, BN, H), lambda i: (0, i, 0)),
            pl.BlockSpec((1, BN, H), lambda i: (1, i, 0)),
            mat(), vec(), vec(), vec(),
            mat(), vec(), vec(), vec(),
        ],
        out_specs=pl.BlockSpec((BN, D), lambda i: (i, 0)),
        out_shape=jax.ShapeDtypeStruct((N, D), jnp.float32),
    )(x, aggr2, aggr2, W1, b1.reshape(1, D), g1.reshape(1, D),
      be1.reshape(1, D), W2, b2.reshape(1, D), g2.reshape(1, D),
      be2.reshape(1, D))


def kernel(x, edge_index, W1, b1, g1, be1, W2, b2, g2, be2):
    x2 = x.reshape(2 * N, H)
    src = edge_index[0]
    dst = edge_index[1]
    aggr2 = _sc_aggregate(x2, src, dst)
    return _tc_mlp(x, aggr2, W1, b1, g1, be1, W2, b2, g2, be2)


# speedup vs baseline: 7.6840x; 1.3386x over previous
import functools

import jax
import jax.numpy as jnp
from jax import lax
from jax.experimental import pallas as pl
from jax.experimental.pallas import tpu as pltpu
from jax.experimental.pallas import tpu_sc as plsc

N = 10000
E = 160000
D = 256
H = 128
NS = 16
EPT = E // (NS)
K = 80
NCHUNK = EPT // K
NB = 2
NP = 10240
RPT = NP // NS


def _sc_aggregate(x2, src, dst):
    mesh = plsc.VectorSubcoreMesh(core_axis_name="c", subcore_axis_name="s")

    @functools.partial(
        pl.kernel,
        mesh=mesh,
        out_type=jax.ShapeDtypeStruct((2, NP, H), jnp.float32),
        scratch_types=(
            [pltpu.VMEM((EPT,), jnp.int32)]
            + [pltpu.VMEM((EPT,), jnp.int32)]
            + [pltpu.VMEM((K,), jnp.int32)] * NB
            + [pltpu.VMEM((K, H), jnp.float32)] * NB
            + [pltpu.VMEM_SHARED((NP, H), jnp.float32)]
            + [pltpu.SemaphoreType.DMA] * NB
        ),
    )
    def agg(x2_hbm, src_hbm, dst_hbm, out_hbm, *scratch):
        idx_all = scratch[0]
        dst_all = scratch[1]
        dst_v = scratch[2:2 + NB]
        rows_v = scratch[2 + NB:2 + 2 * NB]
        accum_sh = scratch[2 + 2 * NB]
        semg = scratch[3 + 2 * NB:3 + 3 * NB]
        c = lax.axis_index("c")
        s = lax.axis_index("s")

        pltpu.sync_copy(src_hbm.at[pl.ds(s * EPT, EPT)], idx_all)
        pltpu.sync_copy(dst_hbm.at[pl.ds(s * EPT, EPT)], dst_all)

        def idx_body(t, carry):
            for j in range(K // 16):
                sl = pl.ds(t * K + j * 16, 16)
                idx_all[sl] = idx_all[sl] * 2 + c
            return carry

        lax.fori_loop(0, NCHUNK, idx_body, 0)

        zvec = jnp.zeros((16,), jnp.float32)
        for i in range(K):
            for j in range(H // 16):
                rows_v[0][i, pl.ds(j * 16, 16)] = zvec

        def zero_body(t, carry):
            pltpu.sync_copy(rows_v[0], accum_sh.at[pl.ds(s * RPT + t * K, K)])
            return carry

        lax.fori_loop(0, RPT // K, zero_body, 0)
        plsc.subcore_barrier()

        def gidx(t):
            return idx_all.at[pl.ds(t * K, K)]

        for b in range(NB):
            pltpu.async_copy(x2_hbm.at[gidx(b)], rows_v[b], semg[b])

        def grp_body(g, carry):
            for b in range(NB):
                t = g * NB + b
                for j in range(K // 16):
                    dst_v[b][pl.ds(j * 16, 16)] = dst_all[pl.ds(t * K + j * 16, 16)]
                pltpu.make_async_copy(x2_hbm.at[gidx(t)], rows_v[b],
                                      semg[b]).wait()
                pltpu.sync_copy(rows_v[b], accum_sh.at[dst_v[b]], add=True)

                @pl.when(t + NB < NCHUNK)
                def _():
                    pltpu.async_copy(x2_hbm.at[gidx(t + NB)], rows_v[b],
                                     semg[b])
            return carry

        NGRP = (NCHUNK - 1) // NB
        lax.fori_loop(0, NGRP, grp_body, 0)

        for t in range(NGRP * NB, NCHUNK):
            b = t % NB
            for j in range(K // 16):
                dst_v[b][pl.ds(j * 16, 16)] = dst_all[pl.ds(t * K + j * 16, 16)]
            pltpu.make_async_copy(x2_hbm.at[gidx(t)], rows_v[b],
                                  semg[b]).wait()
            pltpu.sync_copy(rows_v[b], accum_sh.at[dst_v[b]], add=True)
        plsc.subcore_barrier()

        pltpu.sync_copy(accum_sh.at[pl.ds(s * RPT, RPT)],
                        out_hbm.at[c, pl.ds(s * RPT, RPT)])

    return agg(x2, src, dst)


def _mlp_body(x_ref, a0_ref, a1_ref, W1_ref, b1_ref, g1_ref, be1_ref,
              W2_ref, b2_ref, g2_ref, be2_ref, o_ref):
    h = x_ref[...] + jnp.concatenate([a0_ref[0], a1_ref[0]], axis=1)

    h = jnp.dot(h, W1_ref[...], preferred_element_type=jnp.float32) + b1_ref[...]
    mu = jnp.mean(h, axis=-1, keepdims=True)
    var = jnp.mean((h - mu) ** 2, axis=-1, keepdims=True)
    h = (h - mu) / jnp.sqrt(var + 1e-5) * g1_ref[...] + be1_ref[...]
    h = 0.5 * h * (1.0 + lax.erf(h / jnp.sqrt(2.0).astype(jnp.float32)))

    h = jnp.dot(h, W2_ref[...], preferred_element_type=jnp.float32) + b2_ref[...]
    mu = jnp.mean(h, axis=-1, keepdims=True)
    var = jnp.mean((h - mu) ** 2, axis=-1, keepdims=True)
    h = (h - mu) / jnp.sqrt(var + 1e-5) * g2_ref[...] + be2_ref[...]
    h = 0.5 * h * (1.0 + lax.erf(h / jnp.sqrt(2.0).astype(jnp.float32)))

    o_ref[...] = h


def _tc_mlp(x, aggr2, W1, b1, g1, be1, W2, b2, g2, be2):
    BN = 400
    grid = (N // BN,)
    vec = lambda: pl.BlockSpec((1, D), lambda i: (0, 0))
    mat = lambda: pl.BlockSpec((D, D), lambda i: (0, 0))
    return pl.pallas_call(
        _mlp_body,
        grid=grid,
        in_specs=[
            pl.BlockSpec((BN, D), lambda i: (i, 0)),
            pl.BlockSpec((1, BN, H), lambda i: (0, i, 0)),
            pl.BlockSpec((1, BN, H), lambda i: (1, i, 0)),
            mat(), vec(), vec(), vec(),
            mat(), vec(), vec(), vec(),
        ],
        out_specs=pl.BlockSpec((BN, D), lambda i: (i, 0)),
        out_shape=jax.ShapeDtypeStruct((N, D), jnp.float32),
    )(x, aggr2, aggr2, W1, b1.reshape(1, D), g1.reshape(1, D),
      be1.reshape(1, D), W2, b2.reshape(1, D), g2.reshape(1, D),
      be2.reshape(1, D))


def kernel(x, edge_index, W1, b1, g1, be1, W2, b2, g2, be2):
    x2 = x.reshape(2 * N, H)
    src = edge_index[0]
    dst = edge_index[1]
    aggr2 = _sc_aggregate(x2, src, dst)
    return _tc_mlp(x, aggr2, W1, b1, g1, be1, W2, b2, g2, be2)
